# Initial kernel scaffold; baseline (speedup 1.0000x reference)
import functools

import jax
import jax.numpy as jnp
from jax import lax
from jax.experimental import pallas as pl
from jax.experimental.pallas import tpu as pltpu
from jax.experimental.pallas import tpu_sc as plsc

N_NODES = 10000
N_EDGES = 320000
D_FEAT = 128
D_EDGE = 16

NC = 2
NS = 16
NW = NC * NS
CHUNK = 128
N_CHUNKS = 79
E_PAD = NW * N_CHUNKS * CHUNK
ROWS_PER_TILE = 626
ACC_ROWS = NS * ROWS_PER_TILE


def _node_mm_body(x_ref, w_ref, b_ref, o_ref):
    o_ref[...] = jnp.dot(x_ref[...], w_ref[...],
                         preferred_element_type=jnp.float32) + b_ref[...]


def _node_transform(x, W_node, b_node):
    return pl.pallas_call(
        _node_mm_body,
        out_shape=jax.ShapeDtypeStruct((N_NODES, D_FEAT), jnp.float32),
    )(x, W_node, b_node.reshape(1, D_FEAT))


def _edge_gate_body(ef_ref, w_ref, b_ref, o_ref):
    z = jnp.dot(ef_ref[...], w_ref[...],
                preferred_element_type=jnp.float32) + b_ref[0, 0]
    o_ref[...] = jax.nn.sigmoid(z).reshape(o_ref.shape)


def _edge_gate(ef_pad, W_edge, b_edge):
    efr = ef_pad.reshape(E_PAD // 8, 128)
    W128 = jnp.kron(jnp.eye(8, dtype=jnp.float32), W_edge)
    n_out = E_PAD // 128
    grid = 4
    blk_o = n_out // grid
    blk_i = blk_o * 16
    return pl.pallas_call(
        _edge_gate_body,
        grid=(grid,),
        in_specs=[
            pl.BlockSpec((blk_i, 128), lambda i: (i, 0)),
            pl.BlockSpec((128, 8), lambda i: (0, 0)),
            pl.BlockSpec((1, 1), lambda i: (0, 0), memory_space=pltpu.SMEM),
        ],
        out_specs=pl.BlockSpec((blk_o, 128), lambda i: (i, 0)),
        out_shape=jax.ShapeDtypeStruct((n_out, 128), jnp.float32),
    )(efr, W128, b_edge.reshape(1, 1))


def _sc_body(h_hbm, col_hbm, row_hbm, ew_hbm, z128_hbm, ones16_hbm, z16_hbm,
             p_hbm, cnt_hbm,
             col_v, row_v, ew_v, msgs, ones16, z16, acc, cntacc, sem):
    cid = lax.axis_index("c")
    sid = lax.axis_index("s")
    wid = cid * NS + sid

    pltpu.sync_copy(col_hbm.at[wid], col_v)
    pltpu.sync_copy(row_hbm.at[wid], row_v)
    pltpu.sync_copy(ew_hbm.at[wid], ew_v)
    pltpu.sync_copy(z128_hbm, msgs)
    pltpu.sync_copy(ones16_hbm, ones16)
    pltpu.sync_copy(z16_hbm, z16)

    base = sid * ROWS_PER_TILE
    for k in range(4):
        pltpu.sync_copy(msgs, acc.at[pl.ds(base + k * CHUNK, CHUNK)])
        pltpu.sync_copy(z16, cntacc.at[pl.ds(base + k * CHUNK, CHUNK)])
    rem = ROWS_PER_TILE - 4 * CHUNK
    pltpu.sync_copy(msgs.at[pl.ds(0, rem)],
                    acc.at[pl.ds(base + 4 * CHUNK, rem)])
    pltpu.sync_copy(z16.at[pl.ds(0, rem)],
                    cntacc.at[pl.ds(base + 4 * CHUNK, rem)])
    plsc.subcore_barrier()

    def chunk_step(j, carry):
        pltpu.async_copy(h_hbm.at[col_v.at[j]], msgs, sem).wait()

        def edge_step(e, c):
            w = jnp.full((16,), ew_v[j, e])
            for d in range(D_FEAT // 16):
                sl = pl.ds(d * 16, 16)
                msgs[e, sl] = msgs[e, sl] * w
            return c

        lax.fori_loop(0, CHUNK, edge_step, 0)

        pltpu.sync_copy(msgs, acc.at[row_v.at[j]], add=True)
        pltpu.sync_copy(ones16, cntacc.at[row_v.at[j]], add=True)
        return carry

    lax.fori_loop(0, N_CHUNKS, chunk_step, 0)
    plsc.subcore_barrier()

    for k in range(4):
        pltpu.sync_copy(acc.at[pl.ds(base + k * CHUNK, CHUNK)],
                        p_hbm.at[cid, pl.ds(base + k * CHUNK, CHUNK)])
        pltpu.sync_copy(cntacc.at[pl.ds(base + k * CHUNK, CHUNK)],
                        cnt_hbm.at[cid, pl.ds(base + k * CHUNK, CHUNK)])
    pltpu.sync_copy(acc.at[pl.ds(base + 4 * CHUNK, rem)],
                    p_hbm.at[cid, pl.ds(base + 4 * CHUNK, rem)])
    pltpu.sync_copy(cntacc.at[pl.ds(base + 4 * CHUNK, rem)],
                    cnt_hbm.at[cid, pl.ds(base + 4 * CHUNK, rem)])


def _sc_scatter(h, col3, row3, ew3):
    z128 = jnp.zeros((CHUNK, D_FEAT), jnp.float32)
    ones16 = jnp.ones((CHUNK, 16), jnp.float32)
    z16 = jnp.zeros((CHUNK, 16), jnp.float32)
    mesh = plsc.VectorSubcoreMesh(core_axis_name="c", subcore_axis_name="s")
    k = pl.kernel(
        _sc_body,
        out_type=(
            jax.ShapeDtypeStruct((NC, ACC_ROWS, D_FEAT), jnp.float32),
            jax.ShapeDtypeStruct((NC, ACC_ROWS, 16), jnp.float32),
        ),
        mesh=mesh,
        scratch_types=[
            pltpu.VMEM((N_CHUNKS, CHUNK), jnp.int32),
            pltpu.VMEM((N_CHUNKS, CHUNK), jnp.int32),
            pltpu.VMEM((N_CHUNKS, CHUNK), jnp.float32),
            pltpu.VMEM((CHUNK, D_FEAT), jnp.float32),
            pltpu.VMEM((CHUNK, 16), jnp.float32),
            pltpu.VMEM((CHUNK, 16), jnp.float32),
            pltpu.VMEM_SHARED((ACC_ROWS, D_FEAT), jnp.float32),
            pltpu.VMEM_SHARED((ACC_ROWS, 16), jnp.float32),
            pltpu.SemaphoreType.DMA,
        ],
    )
    return k(h, col3, row3, ew3, z128, ones16, z16)


def _combine_body(p_ref, c_ref, o_ref):
    s = p_ref[0] + p_ref[1]
    cnt = c_ref[0, :, 0:1] + c_ref[1, :, 0:1]
    o_ref[...] = s / jnp.maximum(cnt, 1.0)


def _combine(p, cnt):
    grid = 5
    blk = N_NODES // grid
    return pl.pallas_call(
        _combine_body,
        grid=(grid,),
        in_specs=[
            pl.BlockSpec((NC, blk, D_FEAT), lambda i: (0, i, 0)),
            pl.BlockSpec((NC, blk, 16), lambda i: (0, i, 0)),
        ],
        out_specs=pl.BlockSpec((blk, D_FEAT), lambda i: (i, 0)),
        out_shape=jax.ShapeDtypeStruct((N_NODES, D_FEAT), jnp.float32),
    )(p, cnt)


def kernel(x, edge_index, edge_feature, W_edge, b_edge, W_node, b_node):
    pad = E_PAD - N_EDGES
    row = edge_index[0].astype(jnp.int32)
    col = edge_index[1].astype(jnp.int32)
    row3 = jnp.concatenate(
        [row, jnp.full((pad,), N_NODES, jnp.int32)]).reshape(NW, N_CHUNKS, CHUNK)
    col3 = jnp.concatenate(
        [col, jnp.zeros((pad,), jnp.int32)]).reshape(NW, N_CHUNKS, CHUNK)
    ef_pad = jnp.pad(edge_feature, ((0, pad), (0, 0)))

    h = _node_transform(x, W_node, b_node)
    ew3 = _edge_gate(ef_pad, W_edge, b_edge).reshape(NW, N_CHUNKS, CHUNK)
    p, cnt = _sc_scatter(h, col3, row3, ew3)
    return _combine(p, cnt)


# speedup vs baseline: 2.4323x; 2.4323x over previous
import jax
import jax.numpy as jnp
from jax import lax
from jax.experimental import pallas as pl
from jax.experimental.pallas import tpu as pltpu
from jax.experimental.pallas import tpu_sc as plsc

N_NODES = 10000
N_EDGES = 320000
D_FEAT = 128
D_HALF = 64
D_EDGE = 16

NC = 2
NS = 16
CHUNK = 128
N_CHUNKS = 158
E_PAD = NS * N_CHUNKS * CHUNK
ROWS_PER_TILE = 632
ACC_ROWS = NS * ROWS_PER_TILE


def _node_mm_body(x_ref, w_ref, b_ref, o_ref):
    o_ref[...] = jnp.dot(x_ref[...], w_ref[...],
                         preferred_element_type=jnp.float32) + b_ref[...]


def _node_transform_half(x, W_half, b_half):
    return pl.pallas_call(
        _node_mm_body,
        out_shape=jax.ShapeDtypeStruct((N_NODES, D_HALF), jnp.float32),
    )(x, W_half, b_half.reshape(1, D_HALF))


def _edge_gate_body(ef_ref, w_ref, b_ref, o_ref):
    z = jnp.dot(ef_ref[...], w_ref[...],
                preferred_element_type=jnp.float32) + b_ref[0, 0]
    o_ref[...] = jax.nn.sigmoid(z)


def _edge_gate(ef_pad, W_edge, b_edge):
    efr = ef_pad.reshape(E_PAD // 8, 128)
    n_in = E_PAD // 8
    W128 = jnp.kron(jnp.eye(8, dtype=jnp.float32), W_edge)
    grid = 4
    blk_i = n_in // grid
    return pl.pallas_call(
        _edge_gate_body,
        grid=(grid,),
        in_specs=[
            pl.BlockSpec((blk_i, 128), lambda i: (i, 0)),
            pl.BlockSpec((128, 8), lambda i: (0, 0)),
            pl.BlockSpec((1, 1), lambda i: (0, 0), memory_space=pltpu.SMEM),
        ],
        out_specs=pl.BlockSpec((blk_i, 8), lambda i: (i, 0)),
        out_shape=jax.ShapeDtypeStruct((n_in, 8), jnp.float32),
    )(efr, W128, b_edge.reshape(1, 1))


def _sc_body(h0_hbm, h1_hbm, col_hbm, row_hbm, ew_hbm, z64_hbm, ones16_hbm,
             z16_hbm, p_hbm, cnt_hbm,
             col_v, row_v, ew_v, msgs, ones16, z16, acc, cntacc, sem):
    cid = lax.axis_index("c")
    sid = lax.axis_index("s")

    pltpu.sync_copy(col_hbm.at[sid], col_v)
    pltpu.sync_copy(row_hbm.at[sid], row_v)
    pltpu.sync_copy(ew_hbm.at[sid], ew_v)
    pltpu.sync_copy(z64_hbm, msgs)
    pltpu.sync_copy(ones16_hbm, ones16)
    pltpu.sync_copy(z16_hbm, z16)

    base = sid * ROWS_PER_TILE
    rem = ROWS_PER_TILE - 4 * CHUNK
    for k in range(4):
        pltpu.sync_copy(msgs, acc.at[pl.ds(base + k * CHUNK, CHUNK)])
        pltpu.sync_copy(z16, cntacc.at[pl.ds(base + k * CHUNK, CHUNK)])
    pltpu.sync_copy(msgs.at[pl.ds(0, rem)],
                    acc.at[pl.ds(base + 4 * CHUNK, rem)])
    pltpu.sync_copy(z16.at[pl.ds(0, rem)],
                    cntacc.at[pl.ds(base + 4 * CHUNK, rem)])
    plsc.subcore_barrier()

    def run(h_ref, do_cnt):
        def chunk_step(j, carry):
            pltpu.async_copy(h_ref.at[col_v.at[j]], msgs, sem).wait()

            def group_step(g, c):
                ew16 = ew_v[j, pl.ds(g * 16, 16)]
                for l in range(16):
                    w = jnp.full((16,), ew16[l])
                    e = g * 16 + l
                    for d in range(D_HALF // 16):
                        sl = pl.ds(d * 16, 16)
                        msgs[e, sl] = msgs[e, sl] * w
                return c

            lax.fori_loop(0, CHUNK // 16, group_step, 0)

            pltpu.sync_copy(msgs, acc.at[row_v.at[j]], add=True)
            if do_cnt:
                pltpu.sync_copy(ones16, cntacc.at[row_v.at[j]], add=True)
            return carry

        lax.fori_loop(0, N_CHUNKS, chunk_step, 0)

    @pl.when(cid == 0)
    def _():
        run(h0_hbm, True)

    @pl.when(cid == 1)
    def _():
        run(h1_hbm, False)

    plsc.subcore_barrier()

    for k in range(4):
        pltpu.sync_copy(acc.at[pl.ds(base + k * CHUNK, CHUNK)],
                        p_hbm.at[cid, pl.ds(base + k * CHUNK, CHUNK)])
    pltpu.sync_copy(acc.at[pl.ds(base + 4 * CHUNK, rem)],
                    p_hbm.at[cid, pl.ds(base + 4 * CHUNK, rem)])

    @pl.when(cid == 0)
    def _():
        for k in range(4):
            pltpu.sync_copy(cntacc.at[pl.ds(base + k * CHUNK, CHUNK)],
                            cnt_hbm.at[pl.ds(base + k * CHUNK, CHUNK)])
        pltpu.sync_copy(cntacc.at[pl.ds(base + 4 * CHUNK, rem)],
                        cnt_hbm.at[pl.ds(base + 4 * CHUNK, rem)])


def _sc_scatter(h0, h1, col3, row3, ew3):
    z64 = jnp.zeros((CHUNK, D_HALF), jnp.float32)
    ones16 = jnp.ones((CHUNK, 16), jnp.float32)
    z16 = jnp.zeros((CHUNK, 16), jnp.float32)
    mesh = plsc.VectorSubcoreMesh(core_axis_name="c", subcore_axis_name="s")
    k = pl.kernel(
        _sc_body,
        compiler_params=pltpu.CompilerParams(use_tc_tiling_on_sc=False),
        out_type=(
            jax.ShapeDtypeStruct((NC, ACC_ROWS, D_HALF), jnp.float32),
            jax.ShapeDtypeStruct((ACC_ROWS, 16), jnp.float32),
        ),
        mesh=mesh,
        scratch_types=[
            pltpu.VMEM((N_CHUNKS, CHUNK), jnp.int32),
            pltpu.VMEM((N_CHUNKS, CHUNK), jnp.int32),
            pltpu.VMEM((N_CHUNKS, CHUNK), jnp.float32),
            pltpu.VMEM((CHUNK, D_HALF), jnp.float32),
            pltpu.VMEM((CHUNK, 16), jnp.float32),
            pltpu.VMEM((CHUNK, 16), jnp.float32),
            pltpu.VMEM_SHARED((ACC_ROWS, D_HALF), jnp.float32),
            pltpu.VMEM_SHARED((ACC_ROWS, 16), jnp.float32),
            pltpu.SemaphoreType.DMA,
        ],
    )
    return k(h0, h1, col3, row3, ew3, z64, ones16, z16)


def _combine_body(p_ref, c_ref, o_ref):
    inv = 1.0 / jnp.maximum(c_ref[:, 0:1], 1.0)
    o_ref[:, 0:D_HALF] = p_ref[0] * inv
    o_ref[:, D_HALF:D_FEAT] = p_ref[1] * inv


def _combine(p, cnt):
    grid = 5
    blk = N_NODES // grid
    return pl.pallas_call(
        _combine_body,
        grid=(grid,),
        in_specs=[
            pl.BlockSpec((NC, blk, D_HALF), lambda i: (0, i, 0)),
            pl.BlockSpec((blk, 16), lambda i: (i, 0)),
        ],
        out_specs=pl.BlockSpec((blk, D_FEAT), lambda i: (i, 0)),
        out_shape=jax.ShapeDtypeStruct((N_NODES, D_FEAT), jnp.float32),
    )(p, cnt)


def kernel(x, edge_index, edge_feature, W_edge, b_edge, W_node, b_node):
    pad = E_PAD - N_EDGES
    row = edge_index[0].astype(jnp.int32)
    col = edge_index[1].astype(jnp.int32)
    row3 = jnp.concatenate(
        [row, jnp.full((pad,), N_NODES, jnp.int32)]).reshape(NS, N_CHUNKS, CHUNK)
    col3 = jnp.concatenate(
        [col, jnp.zeros((pad,), jnp.int32)]).reshape(NS, N_CHUNKS, CHUNK)
    ef_pad = jnp.pad(edge_feature, ((0, pad), (0, 0)))

    h0 = _node_transform_half(x, W_node[:, :D_HALF], b_node[:D_HALF])
    h1 = _node_transform_half(x, W_node[:, D_HALF:], b_node[D_HALF:])
    ew3 = _edge_gate(ef_pad, W_edge, b_edge).reshape(NS, N_CHUNKS, CHUNK)
    p, cnt = _sc_scatter(h0, h1, col3, row3, ew3)
    return _combine(p, cnt)


# speedup vs baseline: 2.8084x; 1.1546x over previous
import jax
import jax.numpy as jnp
from jax import lax
from jax.experimental import pallas as pl
from jax.experimental.pallas import tpu as pltpu
from jax.experimental.pallas import tpu_sc as plsc

N_NODES = 10000
N_EDGES = 320000
D_FEAT = 128
D_HALF = 64
D_EDGE = 16

NC = 2
NS = 16
CHUNK = 128
N_CHUNKS = 158
N_CHUNKS_PAD = 160
ROWS_PER_TILE = 632
ACC_ROWS = NS * ROWS_PER_TILE


def _node_mm_body(x_ref, w_ref, b_ref, o_ref):
    o_ref[...] = jnp.dot(x_ref[...], w_ref[...],
                         preferred_element_type=jnp.float32) + b_ref[...]


def _node_transform_half(x, W_half, b_half):
    return pl.pallas_call(
        _node_mm_body,
        out_shape=jax.ShapeDtypeStruct((N_NODES, D_HALF), jnp.float32),
    )(x, W_half, b_half.reshape(1, D_HALF))


def _edge_gate_body(ef_ref, w_ref, b_ref, o_ref):
    z = jnp.dot(ef_ref[...], w_ref[...],
                preferred_element_type=jnp.float32) + b_ref[0, 0]
    o_ref[...] = jax.nn.sigmoid(z)


def _edge_gate(edge_feature, W_edge, b_edge):
    efr = edge_feature.reshape(N_EDGES // 8, 128)
    n_in = N_EDGES // 8
    W128 = jnp.kron(jnp.eye(8, dtype=jnp.float32), W_edge)
    grid = 4
    blk_i = n_in // grid
    return pl.pallas_call(
        _edge_gate_body,
        grid=(grid,),
        in_specs=[
            pl.BlockSpec((blk_i, 128), lambda i: (i, 0)),
            pl.BlockSpec((128, 8), lambda i: (0, 0)),
            pl.BlockSpec((1, 1), lambda i: (0, 0), memory_space=pltpu.SMEM),
        ],
        out_specs=pl.BlockSpec((blk_i, 8), lambda i: (i, 0)),
        out_shape=jax.ShapeDtypeStruct((n_in, 8), jnp.float32),
    )(efr, W128, b_edge.reshape(1, 1))


def _sc_body(h0_hbm, h1_hbm, col_hbm, row_hbm, ew_hbm, p_hbm, cnt_hbm,
             col_v, row_v, ew_v, msgs_a, msgs_b, cnt_tile, acc,
             gsem_a, gsem_b, ssem_a, ssem_b):
    cid = lax.axis_index("c")
    sid = lax.axis_index("s")

    pltpu.sync_copy(col_hbm.at[sid], col_v)
    pltpu.sync_copy(row_hbm.at[sid], row_v)
    pltpu.sync_copy(ew_hbm.at[sid], ew_v)

    zv = jnp.zeros((16,), jnp.float32)
    ov = jnp.ones((16,), jnp.float32)

    def zero_msgs(i, c):
        for d in range(D_HALF // 16):
            msgs_a[i, pl.ds(d * 16, 16)] = zv
            msgs_b[i, pl.ds(d * 16, 16)] = zv
        return c

    lax.fori_loop(0, CHUNK, zero_msgs, 0)

    def zero_cnt(i, c):
        cnt_tile[pl.ds(i * 16, 16)] = zv
        return c

    lax.fori_loop(0, ACC_ROWS // 16, zero_cnt, 0)

    base = sid * ROWS_PER_TILE
    rem = ROWS_PER_TILE - 4 * CHUNK
    for k in range(4):
        pltpu.sync_copy(msgs_a, acc.at[pl.ds(base + k * CHUNK, CHUNK)])
    pltpu.sync_copy(msgs_a.at[pl.ds(0, rem)],
                    acc.at[pl.ds(base + 4 * CHUNK, rem)])
    plsc.subcore_barrier()

    def scale(msgs, j, count):
        def group_step(g, c):
            ew16 = ew_v[j, pl.ds(g * 16, 16)]
            if count:
                row16 = row_v[j, pl.ds(g * 16, 16)]
                cnts, last = plsc.scan_count(row16)
                plsc.addupdate_scatter(cnt_tile, [row16],
                                       cnts.astype(jnp.float32), mask=last)
            for l in range(16):
                w = jnp.full((16,), ew16[l])
                e = g * 16 + l
                for d in range(D_HALF // 16):
                    sl = pl.ds(d * 16, 16)
                    msgs[e, sl] = msgs[e, sl] * w
            return c

        lax.fori_loop(0, CHUNK // 16, group_step, 0)

    def run(h_ref, count):
        pltpu.async_copy(msgs_b, acc.at[row_v.at[0]], ssem_b, add=True)
        pltpu.async_copy(h_ref.at[col_v.at[0]], msgs_a, gsem_a)

        def slot(c, msgs, other, gsem, gsem_o, ssem, ssem_o):
            pltpu.make_async_copy(h_ref.at[col_v.at[c]], msgs, gsem).wait()
            scale(msgs, c, count)
            pltpu.async_copy(msgs, acc.at[row_v.at[c]], ssem, add=True)
            pltpu.make_async_copy(other, acc.at[row_v.at[c]], ssem_o).wait()
            pltpu.async_copy(h_ref.at[col_v.at[c + 1]], other, gsem_o)

        def pair_step(t, carry):
            a = 2 * t
            slot(a, msgs_a, msgs_b, gsem_a, gsem_b, ssem_a, ssem_b)
            slot(a + 1, msgs_b, msgs_a, gsem_b, gsem_a, ssem_b, ssem_a)
            return carry

        lax.fori_loop(0, N_CHUNKS // 2, pair_step, 0)

        pltpu.make_async_copy(h_ref.at[col_v.at[0]], msgs_a, gsem_a).wait()
        pltpu.make_async_copy(msgs_b, acc.at[row_v.at[0]], ssem_b).wait()

    @pl.when(cid == 0)
    def _():
        run(h0_hbm, True)

    @pl.when(cid == 1)
    def _():
        run(h1_hbm, False)

    plsc.subcore_barrier()

    for k in range(4):
        pltpu.sync_copy(acc.at[pl.ds(base + k * CHUNK, CHUNK)],
                        p_hbm.at[cid, pl.ds(base + k * CHUNK, CHUNK)])
    pltpu.sync_copy(acc.at[pl.ds(base + 4 * CHUNK, rem)],
                    p_hbm.at[cid, pl.ds(base + 4 * CHUNK, rem)])

    @pl.when(cid == 0)
    def _():
        pltpu.sync_copy(cnt_tile, cnt_hbm.at[sid])


def _sc_scatter(h0, h1, col3, row3, ew3):
    mesh = plsc.VectorSubcoreMesh(core_axis_name="c", subcore_axis_name="s")
    k = pl.kernel(
        _sc_body,
        compiler_params=pltpu.CompilerParams(use_tc_tiling_on_sc=False,
                                             needs_layout_passes=False),
        out_type=(
            jax.ShapeDtypeStruct((NC, ACC_ROWS, D_HALF), jnp.float32),
            jax.ShapeDtypeStruct((NS, ACC_ROWS), jnp.float32),
        ),
        mesh=mesh,
        scratch_types=[
            pltpu.VMEM((N_CHUNKS_PAD, CHUNK), jnp.int32),
            pltpu.VMEM((N_CHUNKS_PAD, CHUNK), jnp.int32),
            pltpu.VMEM((N_CHUNKS_PAD, CHUNK), jnp.float32),
            pltpu.VMEM((CHUNK, D_HALF), jnp.float32),
            pltpu.VMEM((CHUNK, D_HALF), jnp.float32),
            pltpu.VMEM((ACC_ROWS,), jnp.float32),
            pltpu.VMEM_SHARED((ACC_ROWS, D_HALF), jnp.float32),
            pltpu.SemaphoreType.DMA,
            pltpu.SemaphoreType.DMA,
            pltpu.SemaphoreType.DMA,
            pltpu.SemaphoreType.DMA,
        ],
    )
    return k(h0, h1, col3, row3, ew3)


def _combine_body(p_ref, c_ref, o_ref):
    cnt = jnp.sum(c_ref[...], axis=1, keepdims=True)
    inv = 1.0 / jnp.maximum(cnt, 1.0)
    o_ref[:, 0:D_HALF] = p_ref[0] * inv
    o_ref[:, D_HALF:D_FEAT] = p_ref[1] * inv


def _combine(p, cnt_t):
    grid = 5
    blk = N_NODES // grid
    return pl.pallas_call(
        _combine_body,
        grid=(grid,),
        in_specs=[
            pl.BlockSpec((NC, blk, D_HALF), lambda i: (0, i, 0)),
            pl.BlockSpec((blk, NS), lambda i: (i, 0)),
        ],
        out_specs=pl.BlockSpec((blk, D_FEAT), lambda i: (i, 0)),
        out_shape=jax.ShapeDtypeStruct((N_NODES, D_FEAT), jnp.float32),
    )(p, cnt_t)


def kernel(x, edge_index, edge_feature, W_edge, b_edge, W_node, b_node):
    e_main = NS * N_CHUNKS * CHUNK
    pad = e_main - N_EDGES
    n_dummy = N_CHUNKS_PAD - N_CHUNKS
    row = edge_index[0].astype(jnp.int32)
    col = edge_index[1].astype(jnp.int32)
    row3 = jnp.concatenate(
        [row, jnp.full((pad,), N_NODES, jnp.int32)]).reshape(NS, N_CHUNKS,
                                                             CHUNK)
    row3 = jnp.concatenate(
        [row3, jnp.full((NS, n_dummy, CHUNK), N_NODES, jnp.int32)], axis=1)
    col3 = jnp.concatenate(
        [col, jnp.zeros((pad,), jnp.int32)]).reshape(NS, N_CHUNKS, CHUNK)
    col3 = jnp.concatenate(
        [col3, jnp.zeros((NS, n_dummy, CHUNK), jnp.int32)], axis=1)

    h0 = _node_transform_half(x, W_node[:, :D_HALF], b_node[:D_HALF])
    h1 = _node_transform_half(x, W_node[:, D_HALF:], b_node[D_HALF:])
    ew = _edge_gate(edge_feature, W_edge, b_edge).reshape(N_EDGES)
    ew3 = jnp.pad(ew, (0, pad)).reshape(NS, N_CHUNKS, CHUNK)
    ew3 = jnp.concatenate(
        [ew3, jnp.zeros((NS, n_dummy, CHUNK), jnp.float32)], axis=1)
    p, cnt = _sc_scatter(h0, h1, col3, row3, ew3)
    return _combine(p, cnt.T)


# speedup vs baseline: 3.9164x; 1.3945x over previous
import jax
import jax.numpy as jnp
from jax import lax
from jax.experimental import pallas as pl
from jax.experimental.pallas import tpu as pltpu
from jax.experimental.pallas import tpu_sc as plsc

N_NODES = 10000
N_EDGES = 320000
D_FEAT = 128
D_HALF = 64
D_EDGE = 16

NC = 2
NS = 16
CHUNK = 128
N_CHUNKS = 158
N_CHUNKS_PAD = 160
ROWS_PER_TILE = 632
ACC_ROWS = NS * ROWS_PER_TILE


def _node_mm_body(x_ref, w_ref, b_ref, o_ref):
    o_ref[...] = jnp.dot(x_ref[...], w_ref[...],
                         preferred_element_type=jnp.float32) + b_ref[...]


def _node_transform_half(x, W_half, b_half):
    return pl.pallas_call(
        _node_mm_body,
        out_shape=jax.ShapeDtypeStruct((N_NODES, D_HALF), jnp.float32),
    )(x, W_half, b_half.reshape(1, D_HALF))


def _edge_gate_body(ef_ref, w_ref, b_ref, o_ref):
    z = jnp.dot(ef_ref[...], w_ref[...],
                preferred_element_type=jnp.float32) + b_ref[0, 0]
    o_ref[...] = jax.nn.sigmoid(z)


def _edge_gate(edge_feature, W_edge, b_edge):
    efr = edge_feature.reshape(N_EDGES // 8, 128)
    n_in = N_EDGES // 8
    W128 = jnp.kron(jnp.eye(8, dtype=jnp.float32), W_edge)
    grid = 4
    blk_i = n_in // grid
    return pl.pallas_call(
        _edge_gate_body,
        grid=(grid,),
        in_specs=[
            pl.BlockSpec((blk_i, 128), lambda i: (i, 0)),
            pl.BlockSpec((128, 8), lambda i: (0, 0)),
            pl.BlockSpec((1, 1), lambda i: (0, 0), memory_space=pltpu.SMEM),
        ],
        out_specs=pl.BlockSpec((blk_i, 8), lambda i: (i, 0)),
        out_shape=jax.ShapeDtypeStruct((n_in, 8), jnp.float32),
    )(efr, W128, b_edge.reshape(1, 1))


def _sc_body(h0_hbm, h1_hbm, col_hbm, row_hbm, ew_hbm, p_hbm, cnt_hbm,
             col_v, row_v, ew_v, msgs_a, msgs_b, cnt_tile, acc,
             gsem_a, gsem_b, ssem_a, ssem_b):
    cid = lax.axis_index("c")
    sid = lax.axis_index("s")

    pltpu.sync_copy(col_hbm.at[sid], col_v)
    pltpu.sync_copy(row_hbm.at[sid], row_v)
    pltpu.sync_copy(ew_hbm.at[sid], ew_v)

    zv = jnp.zeros((16,), jnp.float32)
    ov = jnp.ones((16,), jnp.float32)

    def zero_msgs(i, c):
        for d in range(D_HALF // 16):
            msgs_a[i, pl.ds(d * 16, 16)] = zv
            msgs_b[i, pl.ds(d * 16, 16)] = zv
        return c

    lax.fori_loop(0, CHUNK, zero_msgs, 0)

    def zero_cnt(i, c):
        cnt_tile[pl.ds(i * 16, 16)] = zv
        return c

    lax.fori_loop(0, ACC_ROWS // 16, zero_cnt, 0)

    base = sid * ROWS_PER_TILE
    rem = ROWS_PER_TILE - 4 * CHUNK
    for k in range(4):
        pltpu.sync_copy(msgs_a, acc.at[pl.ds(base + k * CHUNK, CHUNK)])
    pltpu.sync_copy(msgs_a.at[pl.ds(0, rem)],
                    acc.at[pl.ds(base + 4 * CHUNK, rem)])
    plsc.subcore_barrier()

    def scale(msgs, j):
        def group_step(g, c):
            ew16 = ew_v[j, pl.ds(g * 16, 16)]
            for l in range(16):
                w = jnp.full((16,), ew16[l])
                e = g * 16 + l
                for d in range(D_HALF // 16):
                    sl = pl.ds(d * 16, 16)
                    msgs[e, sl] = msgs[e, sl] * w
            return c

        lax.fori_loop(0, CHUNK // 16, group_step, 0)

    def run(h_ref):
        pltpu.async_copy(msgs_b, acc.at[row_v.at[0]], ssem_b, add=True)
        pltpu.async_copy(h_ref.at[col_v.at[0]], msgs_a, gsem_a)

        def slot(c, msgs, other, gsem, gsem_o, ssem, ssem_o):
            pltpu.make_async_copy(h_ref.at[col_v.at[c]], msgs, gsem).wait()
            scale(msgs, c)
            pltpu.async_copy(msgs, acc.at[row_v.at[c]], ssem, add=True)
            pltpu.make_async_copy(other, acc.at[row_v.at[c]], ssem_o).wait()
            pltpu.async_copy(h_ref.at[col_v.at[c + 1]], other, gsem_o)

        def pair_step(t, carry):
            a = 2 * t
            slot(a, msgs_a, msgs_b, gsem_a, gsem_b, ssem_a, ssem_b)
            slot(a + 1, msgs_b, msgs_a, gsem_b, gsem_a, ssem_b, ssem_a)
            return carry

        lax.fori_loop(0, N_CHUNKS // 2, pair_step, 0)

        pltpu.make_async_copy(h_ref.at[col_v.at[0]], msgs_a, gsem_a).wait()
        pltpu.make_async_copy(msgs_b, acc.at[row_v.at[0]], ssem_b).wait()

    @pl.when(cid == 0)
    def _():
        run(h0_hbm)

    @pl.when(cid == 1)
    def _():
        run(h1_hbm)

    def count_step(j, carry):
        def cgroup(g, c):
            row16 = row_v[j, pl.ds(g * 16, 16)]
            cnts, last = plsc.scan_count(row16)
            plsc.addupdate_scatter(cnt_tile, [row16],
                                   cnts.astype(jnp.float32), mask=last)
            return c

        lax.fori_loop(0, CHUNK // 16, cgroup, 0)
        return carry

    half = N_CHUNKS // 2
    lax.fori_loop(cid * half, (cid + 1) * half, count_step, 0)
    plsc.subcore_barrier()

    for k in range(4):
        pltpu.sync_copy(acc.at[pl.ds(base + k * CHUNK, CHUNK)],
                        p_hbm.at[cid, pl.ds(base + k * CHUNK, CHUNK)])
    pltpu.sync_copy(acc.at[pl.ds(base + 4 * CHUNK, rem)],
                    p_hbm.at[cid, pl.ds(base + 4 * CHUNK, rem)])

    pltpu.sync_copy(cnt_tile, cnt_hbm.at[cid, sid])


def _sc_scatter(h0, h1, col3, row3, ew3):
    mesh = plsc.VectorSubcoreMesh(core_axis_name="c", subcore_axis_name="s")
    k = pl.kernel(
        _sc_body,
        compiler_params=pltpu.CompilerParams(use_tc_tiling_on_sc=False,
                                             needs_layout_passes=False),
        out_type=(
            jax.ShapeDtypeStruct((NC, ACC_ROWS, D_HALF), jnp.float32),
            jax.ShapeDtypeStruct((NC, NS, ACC_ROWS), jnp.float32),
        ),
        mesh=mesh,
        scratch_types=[
            pltpu.VMEM((N_CHUNKS_PAD, CHUNK), jnp.int32),
            pltpu.VMEM((N_CHUNKS_PAD, CHUNK), jnp.int32),
            pltpu.VMEM((N_CHUNKS_PAD, CHUNK), jnp.float32),
            pltpu.VMEM((CHUNK, D_HALF), jnp.float32),
            pltpu.VMEM((CHUNK, D_HALF), jnp.float32),
            pltpu.VMEM((ACC_ROWS,), jnp.float32),
            pltpu.VMEM_SHARED((ACC_ROWS, D_HALF), jnp.float32),
            pltpu.SemaphoreType.DMA,
            pltpu.SemaphoreType.DMA,
            pltpu.SemaphoreType.DMA,
            pltpu.SemaphoreType.DMA,
        ],
    )
    return k(h0, h1, col3, row3, ew3)


def _combine_body(p_ref, c_ref, o_ref):
    cnt = jnp.sum(c_ref[...], axis=1, keepdims=True)
    inv = 1.0 / jnp.maximum(cnt, 1.0)
    o_ref[:, 0:D_HALF] = p_ref[0] * inv
    o_ref[:, D_HALF:D_FEAT] = p_ref[1] * inv


def _combine(p, cnt_t):
    grid = 5
    blk = N_NODES // grid
    return pl.pallas_call(
        _combine_body,
        grid=(grid,),
        in_specs=[
            pl.BlockSpec((NC, blk, D_HALF), lambda i: (0, i, 0)),
            pl.BlockSpec((blk, NC * NS), lambda i: (i, 0)),
        ],
        out_specs=pl.BlockSpec((blk, D_FEAT), lambda i: (i, 0)),
        out_shape=jax.ShapeDtypeStruct((N_NODES, D_FEAT), jnp.float32),
    )(p, cnt_t)


def kernel(x, edge_index, edge_feature, W_edge, b_edge, W_node, b_node):
    e_main = NS * N_CHUNKS * CHUNK
    pad = e_main - N_EDGES
    n_dummy = N_CHUNKS_PAD - N_CHUNKS
    row = edge_index[0].astype(jnp.int32)
    col = edge_index[1].astype(jnp.int32)
    row3 = jnp.concatenate(
        [row, jnp.full((pad,), N_NODES, jnp.int32)]).reshape(NS, N_CHUNKS,
                                                             CHUNK)
    row3 = jnp.concatenate(
        [row3, jnp.full((NS, n_dummy, CHUNK), N_NODES, jnp.int32)], axis=1)
    col3 = jnp.concatenate(
        [col, jnp.zeros((pad,), jnp.int32)]).reshape(NS, N_CHUNKS, CHUNK)
    col3 = jnp.concatenate(
        [col3, jnp.zeros((NS, n_dummy, CHUNK), jnp.int32)], axis=1)

    h0 = _node_transform_half(x, W_node[:, :D_HALF], b_node[:D_HALF])
    h1 = _node_transform_half(x, W_node[:, D_HALF:], b_node[D_HALF:])
    ew = _edge_gate(edge_feature, W_edge, b_edge).reshape(N_EDGES)
    ew3 = jnp.pad(ew, (0, pad)).reshape(NS, N_CHUNKS, CHUNK)
    ew3 = jnp.concatenate(
        [ew3, jnp.zeros((NS, n_dummy, CHUNK), jnp.float32)], axis=1)
    p, cnt = _sc_scatter(h0, h1, col3, row3, ew3)
    return _combine(p, cnt.reshape(NC * NS, ACC_ROWS).T)
